# baseline (speedup 1.0000x reference)
import numpy as np
import jax
import jax.numpy as jnp
from jax.experimental import pallas as pl
from jax.experimental.pallas import tpu as pltpu

PS = 10
STRIDE = 4
K = 7
T = 23
N = T * T
H = 98
CE = 32
CX = 3
FE = CE * PS * PS
FX = CX * PS * PS

_PREC = jax.lax.Precision.HIGHEST


def _np_consts():
    S = np.zeros((T * PS, H), np.float32)
    for t in range(T):
        for a in range(PS):
            S[t * PS + a, STRIDE * t + a] = 1.0
    s1 = S.sum(axis=0)
    invw = (1.0 / (np.outer(s1, s1) + 1e-10)).astype(np.float32)
    return S, invw


_S_NP, _INVW_NP = _np_consts()


def _dot(a, b, dims):
    return jax.lax.dot_general(a, b, (dims, ((), ())), precision=_PREC,
                               preferred_element_type=jnp.float32)


def _unfold_mm(s, img):
    t1 = _dot(s, img, ((1,), (0,)))
    return _dot(t1, s, ((1,), (1,)))


def _patch_rows(u):
    return u.reshape(T, PS, T, PS).transpose(0, 2, 1, 3).reshape(N, PS * PS)


def _mega(xe_ref, ye_ref, x_ref, lt_ref, yp_ref, s_ref, invw_ref,
          o_ref, xep_s, yep_s, xp_s):
    s = s_ref[...]
    p = PS * PS
    for c in range(CE):
        xep_s[:, c * p:(c + 1) * p] = _patch_rows(_unfold_mm(s, xe_ref[c]))
        yep_s[:, c * p:(c + 1) * p] = _patch_rows(_unfold_mm(s, ye_ref[c]))
    for c in range(CX):
        xp_s[:, c * p:(c + 1) * p] = _patch_rows(_unfold_mm(s, x_ref[c]))
    lt_u = _patch_rows(_unfold_mm(s, lt_ref[0]))

    ye_p = yep_s[...]
    xe_p = xep_s[...]
    g = jax.lax.dot_general(ye_p, xe_p, (((1,), (1,)), ((), ())),
                            preferred_element_type=jnp.float32)
    d2 = (jnp.sum(ye_p ** 2, axis=1, keepdims=True)
          + jnp.sum(xe_p ** 2, axis=1)[None, :]
          - 2.0 * g)
    lt_p = jnp.mean(lt_u, axis=1, keepdims=True)
    lg = (-d2) / jnp.exp(lt_p)

    xp = xp_s[...]
    invw = invw_ref[...]
    for k in range(K):
        m = jnp.max(lg, axis=-1, keepdims=True)
        un = jnp.exp(lg - m)
        w = un / jnp.sum(un, axis=-1, keepdims=True)
        zk = jax.lax.dot_general(w, xp, (((1,), (0,)), ((), ())),
                                 preferred_element_type=jnp.float32)
        z5 = zk.reshape(T, T, CX, PS, PS)
        for c in range(CX):
            zm = z5[:, :, c, :, :].transpose(0, 2, 1, 3).reshape(T * PS, T * PS)
            t1 = _dot(s, zm, ((0,), (0,)))
            v = _dot(t1, s, ((1,), (0,)))
            o_ref[k * CX + c] = v * invw - yp_ref[c]
        if k < K - 1:
            lg = lg + jnp.log(jnp.clip(1.0 - w, 1e-10, None))


def kernel(x, xe, ye, y, log_temp):
    f32 = jnp.float32
    pad = lambda v: jnp.pad(v, ((0, 0), (0, 0), (1, 1), (1, 1)))[0]
    xp_img = pad(x)
    xe_img = pad(xe)
    ye_img = pad(ye)
    yp_img = pad(y)
    lt_img = pad(log_temp)

    out = pl.pallas_call(
        _mega,
        out_shape=jax.ShapeDtypeStruct((K * CX, H, H), f32),
        scratch_shapes=[
            pltpu.VMEM((N, FE), f32),
            pltpu.VMEM((N, FE), f32),
            pltpu.VMEM((N, FX), f32),
        ],
    )(xe_img, ye_img, xp_img, lt_img, yp_img,
      jnp.asarray(_S_NP), jnp.asarray(_INVW_NP))

    z = jnp.concatenate([yp_img[None], out[None]], axis=1)
    return z[..., 1:-1, 1:-1]
